# baseline (speedup 1.0000x reference)
import functools

import jax
import jax.numpy as jnp
from jax import lax
from jax.experimental import pallas as pl
from jax.experimental.pallas import tpu as pltpu
from jax.experimental.pallas import tpu_sc as plsc


def _eye(n):
    a = lax.broadcasted_iota(jnp.int32, (n, n), 0)
    b = lax.broadcasted_iota(jnp.int32, (n, n), 1)
    return (a == b).astype(jnp.float32)


def _mxu_t(x):
    return lax.dot_general(
        x, _eye(x.shape[0]), (((0,), (0,)), ((), ())),
        preferred_element_type=jnp.float32)


def _pack_table(tableT, tailT, NC, NS):
    D, V = tableT.shape
    NW = NC * NS
    full = V // 128
    rem = full % NW
    base_n = full // NW

    mesh = plsc.VectorSubcoreMesh(core_axis_name="c", subcore_axis_name="s")

    @functools.partial(
        pl.kernel,
        out_type=jax.ShapeDtypeStruct((V // 2, 128), jnp.float32),
        mesh=mesh,
        compiler_params=pltpu.CompilerParams(
            use_tc_tiling_on_sc=True, skip_device_barrier=True,
            needs_layout_passes=False,
        ),
        scratch_types=[
            pltpu.VMEM((D, 128), jnp.float32),
            pltpu.VMEM((64, 128), jnp.float32),
        ],
    )
    def pack_kernel(tT_hbm, tail_hbm, out_hbm, slab_v, obuf_v):
        wid = lax.axis_index("s") * NC + lax.axis_index("c")
        nch = jnp.where(wid < rem, base_n + 1, base_n)

        def transpose_rows(nrows):
            def prow(p, carry):
                for u in range(8):
                    q = lax.broadcasted_iota(jnp.int32, (16,), 0) + (u * 16)
                    rv = q & (D - 1)
                    cv = lax.shift_right_logical(q, 6) + 2 * p
                    obuf_v[p, pl.ds(u * 16, 16)] = plsc.load_gather(
                        slab_v, [rv, cv])
                return carry
            lax.fori_loop(0, nrows, prow, 0, unroll=False)

        def body(i, carry):
            c = wid + NW * i
            c0 = pl.multiple_of(c * 128, 128)
            pltpu.sync_copy(tT_hbm.at[:, pl.ds(c0, 128)], slab_v)
            transpose_rows(64)
            pltpu.sync_copy(obuf_v, out_hbm.at[pl.ds(c * 64, 64)])
            return carry

        lax.fori_loop(0, nch, body, 0, unroll=False)

        if V % 128:
            @pl.when(wid == 0)
            def _():
                pltpu.sync_copy(tail_hbm, slab_v)
                transpose_rows(64)
                pltpu.sync_copy(obuf_v, out_hbm.at[pl.ds(V // 2 - 64, 64)])

    return pack_kernel(tableT, tailT)


def _make_gather(B, T, V, D, NC, NS):
    NW = NC * NS
    bpw = B // NW
    npr = T // 2
    groups = npr // 2
    gr = 4 * bpw
    nvec = gr // 16
    assert B % NW == 0 and T % 4 == 0 and groups % 2 == 0 and bpw == 128

    mesh = plsc.VectorSubcoreMesh(core_axis_name="c", subcore_axis_name="s")

    @functools.partial(
        pl.kernel,
        out_type=jax.ShapeDtypeStruct((B * T, D), jnp.float32),
        mesh=mesh,
        compiler_params=pltpu.CompilerParams(
            use_tc_tiling_on_sc=False, skip_device_barrier=True,
            needs_layout_passes=False,
        ),
        scratch_types=[
            pltpu.VMEM((bpw * T,), jnp.int32),
            pltpu.VMEM((gr,), jnp.int32),
            pltpu.VMEM((gr,), jnp.int32),
            pltpu.VMEM((gr,), jnp.int32),
            pltpu.VMEM((gr, D), jnp.float32),
            pltpu.VMEM((gr, D), jnp.float32),
            pltpu.SemaphoreType.DMA,
            pltpu.SemaphoreType.DMA,
        ],
    )
    def gather_kernel(ids_hbm, table_hbm, out_hbm, idx_v, off_v,
                      ig0, ig1, b0, b1, g0, g1):
        igs = (ig0, ig1)
        bufs = (b0, b1)
        gsems = (g0, g1)
        wid = lax.axis_index("s") * NC + lax.axis_index("c")
        wrow = wid * bpw * T
        pltpu.sync_copy(ids_hbm.at[pl.ds(pl.multiple_of(wrow, 8), bpw * T)],
                        idx_v)

        for m in range(nvec):
            k = lax.broadcasted_iota(jnp.int32, (16,), 0) + (m * 16)
            jl = lax.shift_right_logical(k, 8)
            b = lax.shift_right_logical(k & (2 * bpw - 1), 1)
            h = k & 1
            off_v[pl.ds(m * 16, 16)] = b * T + 2 * jl + h

        def build_idx(g, s):
            tadd = 4 * g
            for m in range(nvec):
                ov = off_v[pl.ds(m * 16, 16)] + tadd
                igs[s][pl.ds(m * 16, 16)] = plsc.load_gather(idx_v, [ov])

        def gather_cp(s):
            return pltpu.make_async_copy(
                table_hbm.at[igs[s]], bufs[s], gsems[s])

        def out_write(g, s):
            start = pl.multiple_of(wrow + g * gr, 8)
            pltpu.sync_copy(bufs[s], out_hbm.at[pl.ds(start, gr)])

        build_idx(0, 0)
        gather_cp(0).start()
        build_idx(1, 1)
        gather_cp(1).start()

        def body(m, carry):
            g = 2 * m
            gather_cp(0).wait()
            out_write(g, 0)
            build_idx(g + 2, 0)
            gather_cp(0).start()
            gather_cp(1).wait()
            out_write(g + 1, 1)
            build_idx(g + 3, 1)
            gather_cp(1).start()
            return carry

        lax.fori_loop(0, groups // 2 - 1, body, 0, unroll=False)

        gather_cp(0).wait()
        out_write(groups - 2, 0)
        gather_cp(1).wait()
        out_write(groups - 1, 1)

    return gather_kernel


def _unpack_body(z_ref, o_ref):
    npr = o_ref.shape[0] // 2
    for j in range(npr):
        slab = z_ref[j * 128:(j + 1) * 128, :]
        st = _mxu_t(slab)
        o_ref[2 * j:2 * j + 2, :, :] = st.reshape(2, o_ref.shape[1], 128)


def _unpack(z128, B, T, D):
    nb = B // 128
    rows = T // 2 * 128
    return pl.pallas_call(
        _unpack_body,
        grid=(nb,),
        in_specs=[pl.BlockSpec((rows, 128), lambda i: (i, 0))],
        out_specs=pl.BlockSpec((T, D, 128), lambda i: (0, 0, i)),
        out_shape=jax.ShapeDtypeStruct((T, D, B), jnp.float32),
    )(z128)


def _mask_body(ids_ref, m_ref):
    m_ref[...] = ids_ref[...] != 0


def kernel(ids, table):
    B, T = ids.shape
    V, D = table.shape

    info = plsc.get_sparse_core_info()
    NC, NS = info.num_cores, info.num_subcores

    tableT = table.T
    tailT = lax.slice(tableT, (0, V - 128), (D, V))
    packed = _pack_table(tableT, tailT, NC, NS)
    table_lin = packed.reshape(V, D)

    gather = _make_gather(B, T, V, D, NC, NS)
    z2 = gather(ids.reshape(B * T), table_lin)
    out3 = _unpack(z2.reshape(B * T // 2, 128), B, T, D)
    tokens = jnp.transpose(out3, (2, 0, 1))

    mask = pl.pallas_call(
        _mask_body,
        out_shape=jax.ShapeDtypeStruct((B, T), jnp.bool_),
    )(ids)
    return tokens, mask


# speedup vs baseline: 1.1926x; 1.1926x over previous
import functools

import jax
import jax.numpy as jnp
from jax import lax
from jax.experimental import pallas as pl
from jax.experimental.pallas import tpu as pltpu
from jax.experimental.pallas import tpu_sc as plsc


def _eye(n):
    a = lax.broadcasted_iota(jnp.int32, (n, n), 0)
    b = lax.broadcasted_iota(jnp.int32, (n, n), 1)
    return (a == b).astype(jnp.float32)


def _mxu_t(x):
    return lax.dot_general(
        x, _eye(x.shape[0]), (((0,), (0,)), ((), ())),
        preferred_element_type=jnp.float32)


def _pack_table(tableT, tailT, NC, NS):
    D, V = tableT.shape
    NW = NC * NS
    W = 256
    full = V // W
    rem = full % NW
    base_n = full // NW
    PR = W // 2

    mesh = plsc.VectorSubcoreMesh(core_axis_name="c", subcore_axis_name="s")

    @functools.partial(
        pl.kernel,
        out_type=jax.ShapeDtypeStruct((V // 2, 128), jnp.float32),
        mesh=mesh,
        compiler_params=pltpu.CompilerParams(
            use_tc_tiling_on_sc=True, skip_device_barrier=True,
            needs_layout_passes=False,
        ),
        scratch_types=[
            pltpu.VMEM((D, W), jnp.float32),
            pltpu.VMEM((D, W), jnp.float32),
            pltpu.VMEM((PR, 128), jnp.float32),
            pltpu.VMEM((PR, 128), jnp.float32),
            pltpu.VMEM((D, 128), jnp.float32),
            pltpu.SemaphoreType.DMA,
            pltpu.SemaphoreType.DMA,
            pltpu.SemaphoreType.DMA,
            pltpu.SemaphoreType.DMA,
        ],
    )
    def pack_kernel(tT_hbm, tail_hbm, out_hbm, sl0, sl1, ob0, ob1, tl_v,
                    gi0, gi1, go0, go1):
        slabs = (sl0, sl1)
        obufs = (ob0, ob1)
        gis = (gi0, gi1)
        gos = (go0, go1)
        wid = lax.axis_index("s") * NC + lax.axis_index("c")
        nch = jnp.where(wid < rem, base_n + 1, base_n)

        def cidx(i):
            return wid + NW * i

        def in_cp(i, s):
            c0 = pl.multiple_of(cidx(i) * W, 128)
            return pltpu.make_async_copy(
                tT_hbm.at[:, pl.ds(c0, W)], slabs[s], gis[s])

        def out_cp(i, s):
            start = pl.multiple_of(cidx(i) * PR, 8)
            return pltpu.make_async_copy(
                obufs[s], out_hbm.at[pl.ds(start, PR)], gos[s])

        def transpose_rows(slab, obuf, nrows):
            def prow(p, carry):
                for u in range(8):
                    q = lax.broadcasted_iota(jnp.int32, (16,), 0) + (u * 16)
                    rv = q & (D - 1)
                    cv = lax.shift_right_logical(q, 6) + 2 * p
                    obuf[p, pl.ds(u * 16, 16)] = plsc.load_gather(
                        slab, [rv, cv])
                return carry
            lax.fori_loop(0, nrows, prow, 0, unroll=False)

        in_cp(0, 0).start()
        in_cp(1, 1).start()

        def body2(j, carry):
            for s in range(2):
                i = 2 * j + s

                @pl.when(i < nch)
                def _():
                    in_cp(i, s).wait()

                    @pl.when(i >= 2)
                    def _():
                        out_cp(i - 2, s).wait()
                    transpose_rows(slabs[s], obufs[s], PR)
                    out_cp(i, s).start()

                    @pl.when(i + 2 < nch)
                    def _():
                        in_cp(i + 2, s).start()
            return carry

        lax.fori_loop(0, (base_n + 2) // 2, body2, 0, unroll=False)
        for s in range(2):
            last = jnp.where((nch - 1) % 2 == s, nch - 1, nch - 2)
            out_cp(last, s).wait()

        if V % W:
            @pl.when(wid == 0)
            def _():
                pltpu.sync_copy(tail_hbm, tl_v)
                transpose_rows(tl_v, ob0, 64)
                pltpu.sync_copy(ob0.at[pl.ds(0, 64)],
                                out_hbm.at[pl.ds(V // 2 - 64, 64)])

    return pack_kernel(tableT, tailT)


def _make_gather(B, T, V, D, NC, NS):
    NW = NC * NS
    bpw = B // NW
    npr = T // 2
    groups = npr // 2
    gr = 4 * bpw
    nvec = gr // 16
    assert B % NW == 0 and T % 4 == 0 and groups % 2 == 0 and bpw == 128

    mesh = plsc.VectorSubcoreMesh(core_axis_name="c", subcore_axis_name="s")

    @functools.partial(
        pl.kernel,
        out_type=jax.ShapeDtypeStruct((B * T, D), jnp.float32),
        mesh=mesh,
        compiler_params=pltpu.CompilerParams(
            use_tc_tiling_on_sc=False, skip_device_barrier=True,
            needs_layout_passes=False,
        ),
        scratch_types=[
            pltpu.VMEM((bpw * T,), jnp.int32),
            pltpu.VMEM((gr,), jnp.int32),
            pltpu.VMEM((gr,), jnp.int32),
            pltpu.VMEM((gr,), jnp.int32),
            pltpu.VMEM((gr, D), jnp.float32),
            pltpu.VMEM((gr, D), jnp.float32),
            pltpu.SemaphoreType.DMA,
            pltpu.SemaphoreType.DMA,
        ],
    )
    def gather_kernel(ids_hbm, table_hbm, out_hbm, idx_v, off_v,
                      ig0, ig1, b0, b1, g0, g1):
        igs = (ig0, ig1)
        bufs = (b0, b1)
        gsems = (g0, g1)
        wid = lax.axis_index("s") * NC + lax.axis_index("c")
        wrow = wid * bpw * T
        pltpu.sync_copy(ids_hbm.at[pl.ds(pl.multiple_of(wrow, 8), bpw * T)],
                        idx_v)

        for m in range(nvec):
            k = lax.broadcasted_iota(jnp.int32, (16,), 0) + (m * 16)
            jl = lax.shift_right_logical(k, 8)
            b = lax.shift_right_logical(k & (2 * bpw - 1), 1)
            h = k & 1
            off_v[pl.ds(m * 16, 16)] = b * T + 2 * jl + h

        def build_idx(g, s):
            tadd = 4 * g
            for m in range(nvec):
                ov = off_v[pl.ds(m * 16, 16)] + tadd
                igs[s][pl.ds(m * 16, 16)] = plsc.load_gather(idx_v, [ov])

        def gather_cp(s):
            return pltpu.make_async_copy(
                table_hbm.at[igs[s]], bufs[s], gsems[s])

        def out_write(g, s):
            start = pl.multiple_of(wrow + g * gr, 8)
            pltpu.sync_copy(bufs[s], out_hbm.at[pl.ds(start, gr)])

        build_idx(0, 0)
        gather_cp(0).start()
        build_idx(1, 1)
        gather_cp(1).start()

        def body(m, carry):
            g = 2 * m
            gather_cp(0).wait()
            out_write(g, 0)
            build_idx(g + 2, 0)
            gather_cp(0).start()
            gather_cp(1).wait()
            out_write(g + 1, 1)
            build_idx(g + 3, 1)
            gather_cp(1).start()
            return carry

        lax.fori_loop(0, groups // 2 - 1, body, 0, unroll=False)

        gather_cp(0).wait()
        out_write(groups - 2, 0)
        gather_cp(1).wait()
        out_write(groups - 1, 1)

    return gather_kernel


def _unpack_body(z_ref, o_ref):
    npr = o_ref.shape[0] // 2
    for j in range(npr):
        slab = z_ref[j * 128:(j + 1) * 128, :]
        st = _mxu_t(slab)
        o_ref[2 * j:2 * j + 2, :, :] = st.reshape(2, o_ref.shape[1], 128)


def _unpack(z128, B, T, D):
    nb = B // 128
    rows = T // 2 * 128
    return pl.pallas_call(
        _unpack_body,
        grid=(nb,),
        in_specs=[pl.BlockSpec((rows, 128), lambda i: (i, 0))],
        out_specs=pl.BlockSpec((T, D, 128), lambda i: (0, 0, i)),
        out_shape=jax.ShapeDtypeStruct((T, D, B), jnp.float32),
    )(z128)


def _mask_body(ids_ref, m_ref):
    m_ref[...] = ids_ref[...] != 0


def kernel(ids, table):
    B, T = ids.shape
    V, D = table.shape

    info = plsc.get_sparse_core_info()
    NC, NS = info.num_cores, info.num_subcores

    tableT = table.T
    tailT = lax.slice(tableT, (0, V - 128), (D, V))
    packed = _pack_table(tableT, tailT, NC, NS)
    table_lin = packed.reshape(V, D)

    gather = _make_gather(B, T, V, D, NC, NS)
    z2 = gather(ids.reshape(B * T), table_lin)
    out3 = _unpack(z2.reshape(B * T // 2, 128), B, T, D)
    tokens = jnp.transpose(out3, (2, 0, 1))

    mask = pl.pallas_call(
        _mask_body,
        out_shape=jax.ShapeDtypeStruct((B, T), jnp.bool_),
    )(ids)
    return tokens, mask


# speedup vs baseline: 1.1927x; 1.0001x over previous
import functools

import jax
import jax.numpy as jnp
from jax import lax
from jax.experimental import pallas as pl
from jax.experimental.pallas import tpu as pltpu
from jax.experimental.pallas import tpu_sc as plsc


def _eye(n):
    a = lax.broadcasted_iota(jnp.int32, (n, n), 0)
    b = lax.broadcasted_iota(jnp.int32, (n, n), 1)
    return (a == b).astype(jnp.float32)


def _mxu_t(x):
    return lax.dot_general(
        x, _eye(x.shape[0]), (((0,), (0,)), ((), ())),
        preferred_element_type=jnp.float32)


def _pack_table(tableT, tailT, NC, NS):
    D, V = tableT.shape
    NW = NC * NS
    W = 256
    full = V // W
    rem = full % NW
    base_n = full // NW
    PR = W // 2

    mesh = plsc.VectorSubcoreMesh(core_axis_name="c", subcore_axis_name="s")

    @functools.partial(
        pl.kernel,
        out_type=jax.ShapeDtypeStruct((V // 2, 128), jnp.float32),
        mesh=mesh,
        compiler_params=pltpu.CompilerParams(
            use_tc_tiling_on_sc=True, skip_device_barrier=True,
            needs_layout_passes=False,
        ),
        scratch_types=[
            pltpu.VMEM((D, W + 1), jnp.float32),
            pltpu.VMEM((D, W + 1), jnp.float32),
            pltpu.VMEM((PR, 128), jnp.float32),
            pltpu.VMEM((PR, 128), jnp.float32),
            pltpu.VMEM((D, 129), jnp.float32),
            pltpu.SemaphoreType.DMA,
            pltpu.SemaphoreType.DMA,
            pltpu.SemaphoreType.DMA,
            pltpu.SemaphoreType.DMA,
        ],
    )
    def pack_kernel(tT_hbm, tail_hbm, out_hbm, sl0, sl1, ob0, ob1, tl_v,
                    gi0, gi1, go0, go1):
        slabs = (sl0, sl1)
        obufs = (ob0, ob1)
        gis = (gi0, gi1)
        gos = (go0, go1)
        wid = lax.axis_index("s") * NC + lax.axis_index("c")
        nch = jnp.where(wid < rem, base_n + 1, base_n)

        def cidx(i):
            return wid + NW * i

        def in_cp(i, s):
            c0 = pl.multiple_of(cidx(i) * W, 128)
            return pltpu.make_async_copy(
                tT_hbm.at[:, pl.ds(c0, W)], slabs[s].at[:, pl.ds(0, W)],
                gis[s])

        def out_cp(i, s):
            start = pl.multiple_of(cidx(i) * PR, 8)
            return pltpu.make_async_copy(
                obufs[s], out_hbm.at[pl.ds(start, PR)], gos[s])

        def transpose_rows(slab, obuf, nrows):
            def prow(p, carry):
                for u in range(8):
                    q = lax.broadcasted_iota(jnp.int32, (16,), 0) + (u * 16)
                    rv = q & (D - 1)
                    cv = lax.shift_right_logical(q, 6) + 2 * p
                    obuf[p, pl.ds(u * 16, 16)] = plsc.load_gather(
                        slab, [rv, cv])
                return carry
            lax.fori_loop(0, nrows, prow, 0, unroll=False)

        in_cp(0, 0).start()
        in_cp(1, 1).start()

        def body2(j, carry):
            for s in range(2):
                i = 2 * j + s

                @pl.when(i < nch)
                def _():
                    in_cp(i, s).wait()

                    @pl.when(i >= 2)
                    def _():
                        out_cp(i - 2, s).wait()
                    transpose_rows(slabs[s], obufs[s], PR)
                    out_cp(i, s).start()

                    @pl.when(i + 2 < nch)
                    def _():
                        in_cp(i + 2, s).start()
            return carry

        lax.fori_loop(0, (base_n + 2) // 2, body2, 0, unroll=False)
        for s in range(2):
            last = jnp.where((nch - 1) % 2 == s, nch - 1, nch - 2)
            out_cp(last, s).wait()

        if V % W:
            @pl.when(wid == 0)
            def _():
                pltpu.sync_copy(tail_hbm, tl_v.at[:, pl.ds(0, 128)])
                transpose_rows(tl_v, ob0, 64)
                pltpu.sync_copy(ob0.at[pl.ds(0, 64)],
                                out_hbm.at[pl.ds(V // 2 - 64, 64)])

    return pack_kernel(tableT, tailT)


def _make_gather(B, T, V, D, NC, NS):
    NW = NC * NS
    bpw = B // NW
    npr = T // 2
    groups = npr // 2
    gr = 4 * bpw
    nvec = gr // 16
    assert B % NW == 0 and T % 4 == 0 and groups % 2 == 0 and bpw == 128

    mesh = plsc.VectorSubcoreMesh(core_axis_name="c", subcore_axis_name="s")

    @functools.partial(
        pl.kernel,
        out_type=jax.ShapeDtypeStruct((B * T, D), jnp.float32),
        mesh=mesh,
        compiler_params=pltpu.CompilerParams(
            use_tc_tiling_on_sc=False, skip_device_barrier=True,
            needs_layout_passes=False,
        ),
        scratch_types=[
            pltpu.VMEM((bpw * T,), jnp.int32),
            pltpu.VMEM((gr,), jnp.int32),
            pltpu.VMEM((gr,), jnp.int32),
            pltpu.VMEM((gr,), jnp.int32),
            pltpu.VMEM((gr, D), jnp.float32),
            pltpu.VMEM((gr, D), jnp.float32),
            pltpu.SemaphoreType.DMA,
            pltpu.SemaphoreType.DMA,
        ],
    )
    def gather_kernel(ids_hbm, table_hbm, out_hbm, idx_v, off_v,
                      ig0, ig1, b0, b1, g0, g1):
        igs = (ig0, ig1)
        bufs = (b0, b1)
        gsems = (g0, g1)
        wid = lax.axis_index("s") * NC + lax.axis_index("c")
        wrow = wid * bpw * T
        pltpu.sync_copy(ids_hbm.at[pl.ds(pl.multiple_of(wrow, 8), bpw * T)],
                        idx_v)

        for m in range(nvec):
            k = lax.broadcasted_iota(jnp.int32, (16,), 0) + (m * 16)
            jl = lax.shift_right_logical(k, 8)
            b = lax.shift_right_logical(k & (2 * bpw - 1), 1)
            h = k & 1
            off_v[pl.ds(m * 16, 16)] = b * T + 2 * jl + h

        def build_idx(g, s):
            tadd = 4 * g
            for m in range(nvec):
                ov = off_v[pl.ds(m * 16, 16)] + tadd
                igs[s][pl.ds(m * 16, 16)] = plsc.load_gather(idx_v, [ov])

        def gather_cp(s):
            return pltpu.make_async_copy(
                table_hbm.at[igs[s]], bufs[s], gsems[s])

        def out_write(g, s):
            start = pl.multiple_of(wrow + g * gr, 8)
            pltpu.sync_copy(bufs[s], out_hbm.at[pl.ds(start, gr)])

        build_idx(0, 0)
        gather_cp(0).start()
        build_idx(1, 1)
        gather_cp(1).start()

        def body(m, carry):
            g = 2 * m
            gather_cp(0).wait()
            out_write(g, 0)
            build_idx(g + 2, 0)
            gather_cp(0).start()
            gather_cp(1).wait()
            out_write(g + 1, 1)
            build_idx(g + 3, 1)
            gather_cp(1).start()
            return carry

        lax.fori_loop(0, groups // 2 - 1, body, 0, unroll=False)

        gather_cp(0).wait()
        out_write(groups - 2, 0)
        gather_cp(1).wait()
        out_write(groups - 1, 1)

    return gather_kernel


def _unpack_body(z_ref, o_ref):
    npr = o_ref.shape[0] // 2
    for j in range(npr):
        slab = z_ref[j * 128:(j + 1) * 128, :]
        st = _mxu_t(slab)
        o_ref[2 * j:2 * j + 2, :, :] = st.reshape(2, o_ref.shape[1], 128)


def _unpack(z128, B, T, D):
    nb = B // 128
    rows = T // 2 * 128
    return pl.pallas_call(
        _unpack_body,
        grid=(nb,),
        in_specs=[pl.BlockSpec((rows, 128), lambda i: (i, 0))],
        out_specs=pl.BlockSpec((T, D, 128), lambda i: (0, 0, i)),
        out_shape=jax.ShapeDtypeStruct((T, D, B), jnp.float32),
    )(z128)


def _mask_body(ids_ref, m_ref):
    m_ref[...] = ids_ref[...] != 0


def kernel(ids, table):
    B, T = ids.shape
    V, D = table.shape

    info = plsc.get_sparse_core_info()
    NC, NS = info.num_cores, info.num_subcores

    tableT = table.T
    tailT = lax.slice(tableT, (0, V - 128), (D, V))
    packed = _pack_table(tableT, tailT, NC, NS)
    table_lin = packed.reshape(V, D)

    gather = _make_gather(B, T, V, D, NC, NS)
    z2 = gather(ids.reshape(B * T), table_lin)
    out3 = _unpack(z2.reshape(B * T // 2, 128), B, T, D)
    tokens = jnp.transpose(out3, (2, 0, 1))

    mask = pl.pallas_call(
        _mask_body,
        out_shape=jax.ShapeDtypeStruct((B, T), jnp.bool_),
    )(ids)
    return tokens, mask


# speedup vs baseline: 2.7053x; 2.2682x over previous
import functools

import jax
import jax.numpy as jnp
from jax import lax
from jax.experimental import pallas as pl
from jax.experimental.pallas import tpu as pltpu
from jax.experimental.pallas import tpu_sc as plsc


def _eye(n):
    a = lax.broadcasted_iota(jnp.int32, (n, n), 0)
    b = lax.broadcasted_iota(jnp.int32, (n, n), 1)
    return (a == b).astype(jnp.float32)


def _mxu_t(x):
    return lax.dot_general(
        x, _eye(x.shape[0]), (((0,), (0,)), ((), ())),
        preferred_element_type=jnp.float32)


def _tc_pack_body(tT_ref, o_ref):
    D = tT_ref.shape[0]
    W = tT_ref.shape[1]
    i = pl.program_id(0)
    nb = pl.num_programs(0)

    @pl.when(i < nb - 1)
    def _():
        x = tT_ref[...]
        for k in range(W // 256):
            m = jnp.concatenate(
                [x[:, k * 256:k * 256 + 128],
                 x[:, k * 256 + 128:k * 256 + 256]], axis=0)
            o_ref[k * 128:(k + 1) * 128, :] = _mxu_t(m)

    @pl.when(i == nb - 1)
    def _():
        x = tT_ref[:, 0:64]
        xt = _mxu_t(x)
        y = xt.reshape(32, 2, D)
        o_ref[0:32, :] = jnp.concatenate([y[:, 0, :], y[:, 1, :]], axis=1)


def _tc_pack_table(tableT, W=1536):
    D, V = tableT.shape
    full = (V // 256) * 256
    nb = full // W + 1
    assert full % W == 0 and W % 256 == 0
    return pl.pallas_call(
        _tc_pack_body,
        grid=(nb,),
        in_specs=[pl.BlockSpec((D, W), lambda i: (0, i))],
        out_specs=pl.BlockSpec((W * D // 128, 128), lambda i: (i, 0)),
        out_shape=jax.ShapeDtypeStruct((V * D // 128, 128), jnp.float32),
    )(tableT)


def _pack_table(tableT, tailT, NC, NS):
    D, V = tableT.shape
    NW = NC * NS
    W = 256
    full = V // W
    rem = full % NW
    base_n = full // NW
    PR = W // 2

    mesh = plsc.VectorSubcoreMesh(core_axis_name="c", subcore_axis_name="s")

    @functools.partial(
        pl.kernel,
        out_type=jax.ShapeDtypeStruct((V // 2, 128), jnp.float32),
        mesh=mesh,
        compiler_params=pltpu.CompilerParams(
            use_tc_tiling_on_sc=True, skip_device_barrier=True,
            needs_layout_passes=False,
        ),
        scratch_types=[
            pltpu.VMEM((D, W + 1), jnp.float32),
            pltpu.VMEM((D, W + 1), jnp.float32),
            pltpu.VMEM((PR, 128), jnp.float32),
            pltpu.VMEM((PR, 128), jnp.float32),
            pltpu.VMEM((D, 129), jnp.float32),
            pltpu.SemaphoreType.DMA,
            pltpu.SemaphoreType.DMA,
            pltpu.SemaphoreType.DMA,
            pltpu.SemaphoreType.DMA,
        ],
    )
    def pack_kernel(tT_hbm, tail_hbm, out_hbm, sl0, sl1, ob0, ob1, tl_v,
                    gi0, gi1, go0, go1):
        slabs = (sl0, sl1)
        obufs = (ob0, ob1)
        gis = (gi0, gi1)
        gos = (go0, go1)
        wid = lax.axis_index("s") * NC + lax.axis_index("c")
        nch = jnp.where(wid < rem, base_n + 1, base_n)

        def cidx(i):
            return wid + NW * i

        def in_cp(i, s):
            c0 = pl.multiple_of(cidx(i) * W, 128)
            return pltpu.make_async_copy(
                tT_hbm.at[:, pl.ds(c0, W)], slabs[s].at[:, pl.ds(0, W)],
                gis[s])

        def out_cp(i, s):
            start = pl.multiple_of(cidx(i) * PR, 8)
            return pltpu.make_async_copy(
                obufs[s], out_hbm.at[pl.ds(start, PR)], gos[s])

        def transpose_rows(slab, obuf, nrows):
            def prow(p, carry):
                for u in range(8):
                    q = lax.broadcasted_iota(jnp.int32, (16,), 0) + (u * 16)
                    rv = q & (D - 1)
                    cv = lax.shift_right_logical(q, 6) + 2 * p
                    obuf[p, pl.ds(u * 16, 16)] = plsc.load_gather(
                        slab, [rv, cv])
                return carry
            lax.fori_loop(0, nrows, prow, 0, unroll=False)

        in_cp(0, 0).start()
        in_cp(1, 1).start()

        def body2(j, carry):
            for s in range(2):
                i = 2 * j + s

                @pl.when(i < nch)
                def _():
                    in_cp(i, s).wait()

                    @pl.when(i >= 2)
                    def _():
                        out_cp(i - 2, s).wait()
                    transpose_rows(slabs[s], obufs[s], PR)
                    out_cp(i, s).start()

                    @pl.when(i + 2 < nch)
                    def _():
                        in_cp(i + 2, s).start()
            return carry

        lax.fori_loop(0, (base_n + 2) // 2, body2, 0, unroll=False)
        for s in range(2):
            last = jnp.where((nch - 1) % 2 == s, nch - 1, nch - 2)
            out_cp(last, s).wait()

        if V % W:
            @pl.when(wid == 0)
            def _():
                pltpu.sync_copy(tail_hbm, tl_v.at[:, pl.ds(0, 128)])
                transpose_rows(tl_v, ob0, 64)
                pltpu.sync_copy(ob0.at[pl.ds(0, 64)],
                                out_hbm.at[pl.ds(V // 2 - 64, 64)])

    return pack_kernel(tableT, tailT)


def _make_gather(B, T, V, D, NC, NS):
    NW = NC * NS
    bpw = B // NW
    npr = T // 2
    groups = npr // 2
    gr = 4 * bpw
    nvec = gr // 16
    assert B % NW == 0 and T % 4 == 0 and groups % 2 == 0 and bpw == 128

    mesh = plsc.VectorSubcoreMesh(core_axis_name="c", subcore_axis_name="s")

    @functools.partial(
        pl.kernel,
        out_type=jax.ShapeDtypeStruct((B * T, D), jnp.float32),
        mesh=mesh,
        compiler_params=pltpu.CompilerParams(
            use_tc_tiling_on_sc=False, skip_device_barrier=True,
            needs_layout_passes=False,
        ),
        scratch_types=[
            pltpu.VMEM((bpw * T,), jnp.int32),
            pltpu.VMEM((gr,), jnp.int32),
            pltpu.VMEM((gr,), jnp.int32),
            pltpu.VMEM((gr,), jnp.int32),
            pltpu.VMEM((gr, D), jnp.float32),
            pltpu.VMEM((gr, D), jnp.float32),
            pltpu.SemaphoreType.DMA,
            pltpu.SemaphoreType.DMA,
        ],
    )
    def gather_kernel(ids_hbm, table_hbm, out_hbm, idx_v, off_v,
                      ig0, ig1, b0, b1, g0, g1):
        igs = (ig0, ig1)
        bufs = (b0, b1)
        gsems = (g0, g1)
        wid = lax.axis_index("s") * NC + lax.axis_index("c")
        wrow = wid * bpw * T
        pltpu.sync_copy(ids_hbm.at[pl.ds(pl.multiple_of(wrow, 8), bpw * T)],
                        idx_v)

        for m in range(nvec):
            k = lax.broadcasted_iota(jnp.int32, (16,), 0) + (m * 16)
            jl = lax.shift_right_logical(k, 8)
            b = lax.shift_right_logical(k & (2 * bpw - 1), 1)
            h = k & 1
            off_v[pl.ds(m * 16, 16)] = b * T + 2 * jl + h

        full = (V // 256) * 256

        def build_idx(g, s):
            tadd = 4 * g
            for m in range(nvec):
                ov = off_v[pl.ds(m * 16, 16)] + tadd
                v = plsc.load_gather(idx_v, [ov])
                pi = (v & (-256)) + ((v & 127) << 1) + (
                    lax.shift_right_logical(v, 7) & 1)
                igs[s][pl.ds(m * 16, 16)] = jnp.where(v < full, pi, v)

        def gather_cp(s):
            return pltpu.make_async_copy(
                table_hbm.at[igs[s]], bufs[s], gsems[s])

        def out_write(g, s):
            start = pl.multiple_of(wrow + g * gr, 8)
            pltpu.sync_copy(bufs[s], out_hbm.at[pl.ds(start, gr)])

        build_idx(0, 0)
        gather_cp(0).start()
        build_idx(1, 1)
        gather_cp(1).start()

        def body(m, carry):
            g = 2 * m
            gather_cp(0).wait()
            out_write(g, 0)
            build_idx(g + 2, 0)
            gather_cp(0).start()
            gather_cp(1).wait()
            out_write(g + 1, 1)
            build_idx(g + 3, 1)
            gather_cp(1).start()
            return carry

        lax.fori_loop(0, groups // 2 - 1, body, 0, unroll=False)

        gather_cp(0).wait()
        out_write(groups - 2, 0)
        gather_cp(1).wait()
        out_write(groups - 1, 1)

    return gather_kernel


def _unpack_body(z_ref, o_ref):
    npr = o_ref.shape[0] // 2
    for j in range(npr):
        slab = z_ref[j * 128:(j + 1) * 128, :]
        st = _mxu_t(slab)
        o_ref[2 * j:2 * j + 2, :, :] = st.reshape(2, o_ref.shape[1], 128)


def _unpack(z128, B, T, D):
    nb = B // 128
    rows = T // 2 * 128
    return pl.pallas_call(
        _unpack_body,
        grid=(nb,),
        in_specs=[pl.BlockSpec((rows, 128), lambda i: (i, 0))],
        out_specs=pl.BlockSpec((T, D, 128), lambda i: (0, 0, i)),
        out_shape=jax.ShapeDtypeStruct((T, D, B), jnp.float32),
    )(z128)


def _mask_body(ids_ref, m_ref):
    m_ref[...] = ids_ref[...] != 0


def kernel(ids, table):
    B, T = ids.shape
    V, D = table.shape

    info = plsc.get_sparse_core_info()
    NC, NS = info.num_cores, info.num_subcores

    packed = _tc_pack_table(table.T)
    table_lin = packed.reshape(V, D)

    gather = _make_gather(B, T, V, D, NC, NS)
    z2 = gather(ids.reshape(B * T), table_lin)
    out3 = _unpack(z2.reshape(B * T // 2, 128), B, T, D)
    tokens = jnp.transpose(out3, (2, 0, 1))

    mask = pl.pallas_call(
        _mask_body,
        out_shape=jax.ShapeDtypeStruct((B, T), jnp.bool_),
    )(ids)
    return tokens, mask


# speedup vs baseline: 3.1869x; 1.1780x over previous
import functools

import jax
import jax.numpy as jnp
from jax import lax
from jax.experimental import pallas as pl
from jax.experimental.pallas import tpu as pltpu
from jax.experimental.pallas import tpu_sc as plsc


def _eye(n):
    a = lax.broadcasted_iota(jnp.int32, (n, n), 0)
    b = lax.broadcasted_iota(jnp.int32, (n, n), 1)
    return (a == b).astype(jnp.float32)


def _mxu_t(x):
    return lax.dot_general(
        x, _eye(x.shape[0]), (((0,), (0,)), ((), ())),
        preferred_element_type=jnp.float32)


def _tc_pack_body(tT_ref, o_ref):
    D = tT_ref.shape[0]
    W = tT_ref.shape[1]
    i = pl.program_id(0)
    nb = pl.num_programs(0)

    @pl.when(i < nb - 1)
    def _():
        x = tT_ref[...]
        for k in range(W // 256):
            m = jnp.concatenate(
                [x[:, k * 256:k * 256 + 128],
                 x[:, k * 256 + 128:k * 256 + 256]], axis=0)
            o_ref[k * 128:(k + 1) * 128, :] = _mxu_t(m)

    @pl.when(i == nb - 1)
    def _():
        x = tT_ref[:, 0:64]
        xt = _mxu_t(x)
        y = xt.reshape(32, 2, D)
        o_ref[0:32, :] = jnp.concatenate([y[:, 0, :], y[:, 1, :]], axis=1)


def _tc_pack_table(tableT, W=2304):
    D, V = tableT.shape
    full = (V // 256) * 256
    nb = full // W + 1
    assert full % W == 0 and W % 256 == 0
    return pl.pallas_call(
        _tc_pack_body,
        grid=(nb,),
        in_specs=[pl.BlockSpec((D, W), lambda i: (0, i))],
        out_specs=pl.BlockSpec((W * D // 128, 128), lambda i: (i, 0)),
        out_shape=jax.ShapeDtypeStruct((V * D // 128, 128), jnp.float32),
    )(tableT)


def _pack_table(tableT, tailT, NC, NS):
    D, V = tableT.shape
    NW = NC * NS
    W = 256
    full = V // W
    rem = full % NW
    base_n = full // NW
    PR = W // 2

    mesh = plsc.VectorSubcoreMesh(core_axis_name="c", subcore_axis_name="s")

    @functools.partial(
        pl.kernel,
        out_type=jax.ShapeDtypeStruct((V // 2, 128), jnp.float32),
        mesh=mesh,
        compiler_params=pltpu.CompilerParams(
            use_tc_tiling_on_sc=True, skip_device_barrier=True,
            needs_layout_passes=False,
        ),
        scratch_types=[
            pltpu.VMEM((D, W + 1), jnp.float32),
            pltpu.VMEM((D, W + 1), jnp.float32),
            pltpu.VMEM((PR, 128), jnp.float32),
            pltpu.VMEM((PR, 128), jnp.float32),
            pltpu.VMEM((D, 129), jnp.float32),
            pltpu.SemaphoreType.DMA,
            pltpu.SemaphoreType.DMA,
            pltpu.SemaphoreType.DMA,
            pltpu.SemaphoreType.DMA,
        ],
    )
    def pack_kernel(tT_hbm, tail_hbm, out_hbm, sl0, sl1, ob0, ob1, tl_v,
                    gi0, gi1, go0, go1):
        slabs = (sl0, sl1)
        obufs = (ob0, ob1)
        gis = (gi0, gi1)
        gos = (go0, go1)
        wid = lax.axis_index("s") * NC + lax.axis_index("c")
        nch = jnp.where(wid < rem, base_n + 1, base_n)

        def cidx(i):
            return wid + NW * i

        def in_cp(i, s):
            c0 = pl.multiple_of(cidx(i) * W, 128)
            return pltpu.make_async_copy(
                tT_hbm.at[:, pl.ds(c0, W)], slabs[s].at[:, pl.ds(0, W)],
                gis[s])

        def out_cp(i, s):
            start = pl.multiple_of(cidx(i) * PR, 8)
            return pltpu.make_async_copy(
                obufs[s], out_hbm.at[pl.ds(start, PR)], gos[s])

        def transpose_rows(slab, obuf, nrows):
            def prow(p, carry):
                for u in range(8):
                    q = lax.broadcasted_iota(jnp.int32, (16,), 0) + (u * 16)
                    rv = q & (D - 1)
                    cv = lax.shift_right_logical(q, 6) + 2 * p
                    obuf[p, pl.ds(u * 16, 16)] = plsc.load_gather(
                        slab, [rv, cv])
                return carry
            lax.fori_loop(0, nrows, prow, 0, unroll=False)

        in_cp(0, 0).start()
        in_cp(1, 1).start()

        def body2(j, carry):
            for s in range(2):
                i = 2 * j + s

                @pl.when(i < nch)
                def _():
                    in_cp(i, s).wait()

                    @pl.when(i >= 2)
                    def _():
                        out_cp(i - 2, s).wait()
                    transpose_rows(slabs[s], obufs[s], PR)
                    out_cp(i, s).start()

                    @pl.when(i + 2 < nch)
                    def _():
                        in_cp(i + 2, s).start()
            return carry

        lax.fori_loop(0, (base_n + 2) // 2, body2, 0, unroll=False)
        for s in range(2):
            last = jnp.where((nch - 1) % 2 == s, nch - 1, nch - 2)
            out_cp(last, s).wait()

        if V % W:
            @pl.when(wid == 0)
            def _():
                pltpu.sync_copy(tail_hbm, tl_v.at[:, pl.ds(0, 128)])
                transpose_rows(tl_v, ob0, 64)
                pltpu.sync_copy(ob0.at[pl.ds(0, 64)],
                                out_hbm.at[pl.ds(V // 2 - 64, 64)])

    return pack_kernel(tableT, tailT)


def _make_gather(B, T, V, D, NC, NS):
    NW = NC * NS
    bpw = B // NW
    npr = T // 2
    groups = npr // 2
    gr = 4 * bpw
    nvec = gr // 16
    assert B % NW == 0 and T % 4 == 0 and groups % 2 == 0 and bpw == 128

    mesh = plsc.VectorSubcoreMesh(core_axis_name="c", subcore_axis_name="s")

    @functools.partial(
        pl.kernel,
        out_type=jax.ShapeDtypeStruct((B * T, D), jnp.float32),
        mesh=mesh,
        compiler_params=pltpu.CompilerParams(
            use_tc_tiling_on_sc=False, skip_device_barrier=True,
            needs_layout_passes=False,
        ),
        scratch_types=[
            pltpu.VMEM((bpw * T,), jnp.int32),
            pltpu.VMEM((gr,), jnp.int32),
            pltpu.VMEM((gr,), jnp.int32),
            pltpu.VMEM((gr,), jnp.int32),
            pltpu.VMEM((gr, D), jnp.float32),
            pltpu.VMEM((gr, D), jnp.float32),
            pltpu.SemaphoreType.DMA,
            pltpu.SemaphoreType.DMA,
        ],
    )
    def gather_kernel(ids_hbm, table_hbm, out_hbm, idx_v, off_v,
                      ig0, ig1, b0, b1, g0, g1):
        igs = (ig0, ig1)
        bufs = (b0, b1)
        gsems = (g0, g1)
        wid = lax.axis_index("s") * NC + lax.axis_index("c")
        wrow = wid * bpw * T
        pltpu.sync_copy(ids_hbm.at[pl.ds(pl.multiple_of(wrow, 8), bpw * T)],
                        idx_v)

        for m in range(nvec):
            k = lax.broadcasted_iota(jnp.int32, (16,), 0) + (m * 16)
            jl = lax.shift_right_logical(k, 8)
            b = lax.shift_right_logical(k & (2 * bpw - 1), 1)
            h = k & 1
            off_v[pl.ds(m * 16, 16)] = b * T + 2 * jl + h

        full = (V // 256) * 256

        def build_idx(g, s):
            tadd = 4 * g
            for m in range(nvec):
                ov = off_v[pl.ds(m * 16, 16)] + tadd
                v = plsc.load_gather(idx_v, [ov])
                pi = (v & (-256)) + ((v & 127) << 1) + (
                    lax.shift_right_logical(v, 7) & 1)
                igs[s][pl.ds(m * 16, 16)] = jnp.where(v < full, pi, v)

        def gather_cp(s):
            return pltpu.make_async_copy(
                table_hbm.at[igs[s]], bufs[s], gsems[s])

        def out_write(g, s):
            start = pl.multiple_of(wrow + g * gr, 8)
            pltpu.sync_copy(bufs[s], out_hbm.at[pl.ds(start, gr)])

        build_idx(0, 0)
        gather_cp(0).start()
        build_idx(1, 1)
        gather_cp(1).start()

        def body(m, carry):
            g = 2 * m
            gather_cp(0).wait()
            out_write(g, 0)
            build_idx(g + 2, 0)
            gather_cp(0).start()
            gather_cp(1).wait()
            out_write(g + 1, 1)
            build_idx(g + 3, 1)
            gather_cp(1).start()
            return carry

        lax.fori_loop(0, groups // 2 - 1, body, 0, unroll=False)

        gather_cp(0).wait()
        out_write(groups - 2, 0)
        gather_cp(1).wait()
        out_write(groups - 1, 1)

    return gather_kernel


def _unpack_body(z_ref, o_ref):
    npr = o_ref.shape[0] // 2
    for j in range(npr):
        slab = z_ref[j * 128:(j + 1) * 128, :]
        st = _mxu_t(slab)
        o_ref[2 * j:2 * j + 2, :, :] = st.reshape(2, o_ref.shape[1], 128)


def _unpack(z128, B, T, D):
    nb = B // 128
    rows = T // 2 * 128
    return pl.pallas_call(
        _unpack_body,
        grid=(nb,),
        in_specs=[pl.BlockSpec((rows, 128), lambda i: (i, 0))],
        out_specs=pl.BlockSpec((T, D, 128), lambda i: (0, 0, i)),
        out_shape=jax.ShapeDtypeStruct((T, D, B), jnp.float32),
    )(z128)


def _mask_body(ids_ref, m_ref):
    m_ref[...] = ids_ref[...] != 0


def kernel(ids, table):
    B, T = ids.shape
    V, D = table.shape

    info = plsc.get_sparse_core_info()
    NC, NS = info.num_cores, info.num_subcores

    packed = _tc_pack_table(table.T)
    table_lin = packed.reshape(V, D)

    gather = _make_gather(B, T, V, D, NC, NS)
    z2 = gather(ids.reshape(B * T), table_lin)
    out3 = _unpack(z2.reshape(B * T // 2, 128), B, T, D)
    tokens = jnp.transpose(out3, (2, 0, 1))

    mask = pl.pallas_call(
        _mask_body,
        out_shape=jax.ShapeDtypeStruct((B, T), jnp.bool_),
    )(ids)
    return tokens, mask


# speedup vs baseline: 3.8702x; 1.2144x over previous
import functools

import jax
import jax.numpy as jnp
from jax import lax
from jax.experimental import pallas as pl
from jax.experimental.pallas import tpu as pltpu
from jax.experimental.pallas import tpu_sc as plsc


def _eye(n):
    a = lax.broadcasted_iota(jnp.int32, (n, n), 0)
    b = lax.broadcasted_iota(jnp.int32, (n, n), 1)
    return (a == b).astype(jnp.float32)


def _mxu_t(x):
    return lax.dot_general(
        x, _eye(x.shape[0]), (((0,), (0,)), ((), ())),
        preferred_element_type=jnp.float32)


def _tc_pack_body(tT_ref, o_ref):
    D = tT_ref.shape[0]
    W = tT_ref.shape[1]
    i = pl.program_id(0)
    nb = pl.num_programs(0)

    @pl.when(i < nb - 1)
    def _():
        x = tT_ref[...]
        for k in range(W // 256):
            m = jnp.concatenate(
                [x[:, k * 256:k * 256 + 128],
                 x[:, k * 256 + 128:k * 256 + 256]], axis=0)
            o_ref[k * 128:(k + 1) * 128, :] = _mxu_t(m)

    @pl.when(i == nb - 1)
    def _():
        x = tT_ref[:, 0:64]
        xt = _mxu_t(x)
        y = xt.reshape(32, 2, D)
        o_ref[0:32, :] = jnp.concatenate([y[:, 0, :], y[:, 1, :]], axis=1)


def _tc_pack_table(tableT, W=4608):
    D, V = tableT.shape
    full = (V // 256) * 256
    nb = full // W + 1
    assert full % W == 0 and W % 256 == 0
    return pl.pallas_call(
        _tc_pack_body,
        grid=(nb,),
        in_specs=[pl.BlockSpec((D, W), lambda i: (0, i))],
        out_specs=pl.BlockSpec((W * D // 128, 128), lambda i: (i, 0)),
        out_shape=jax.ShapeDtypeStruct((V * D // 128, 128), jnp.float32),
    )(tableT)


def _pack_table(tableT, tailT, NC, NS):
    D, V = tableT.shape
    NW = NC * NS
    W = 256
    full = V // W
    rem = full % NW
    base_n = full // NW
    PR = W // 2

    mesh = plsc.VectorSubcoreMesh(core_axis_name="c", subcore_axis_name="s")

    @functools.partial(
        pl.kernel,
        out_type=jax.ShapeDtypeStruct((V // 2, 128), jnp.float32),
        mesh=mesh,
        compiler_params=pltpu.CompilerParams(
            use_tc_tiling_on_sc=True, skip_device_barrier=True,
            needs_layout_passes=False,
        ),
        scratch_types=[
            pltpu.VMEM((D, W + 1), jnp.float32),
            pltpu.VMEM((D, W + 1), jnp.float32),
            pltpu.VMEM((PR, 128), jnp.float32),
            pltpu.VMEM((PR, 128), jnp.float32),
            pltpu.VMEM((D, 129), jnp.float32),
            pltpu.SemaphoreType.DMA,
            pltpu.SemaphoreType.DMA,
            pltpu.SemaphoreType.DMA,
            pltpu.SemaphoreType.DMA,
        ],
    )
    def pack_kernel(tT_hbm, tail_hbm, out_hbm, sl0, sl1, ob0, ob1, tl_v,
                    gi0, gi1, go0, go1):
        slabs = (sl0, sl1)
        obufs = (ob0, ob1)
        gis = (gi0, gi1)
        gos = (go0, go1)
        wid = lax.axis_index("s") * NC + lax.axis_index("c")
        nch = jnp.where(wid < rem, base_n + 1, base_n)

        def cidx(i):
            return wid + NW * i

        def in_cp(i, s):
            c0 = pl.multiple_of(cidx(i) * W, 128)
            return pltpu.make_async_copy(
                tT_hbm.at[:, pl.ds(c0, W)], slabs[s].at[:, pl.ds(0, W)],
                gis[s])

        def out_cp(i, s):
            start = pl.multiple_of(cidx(i) * PR, 8)
            return pltpu.make_async_copy(
                obufs[s], out_hbm.at[pl.ds(start, PR)], gos[s])

        def transpose_rows(slab, obuf, nrows):
            def prow(p, carry):
                for u in range(8):
                    q = lax.broadcasted_iota(jnp.int32, (16,), 0) + (u * 16)
                    rv = q & (D - 1)
                    cv = lax.shift_right_logical(q, 6) + 2 * p
                    obuf[p, pl.ds(u * 16, 16)] = plsc.load_gather(
                        slab, [rv, cv])
                return carry
            lax.fori_loop(0, nrows, prow, 0, unroll=False)

        in_cp(0, 0).start()
        in_cp(1, 1).start()

        def body2(j, carry):
            for s in range(2):
                i = 2 * j + s

                @pl.when(i < nch)
                def _():
                    in_cp(i, s).wait()

                    @pl.when(i >= 2)
                    def _():
                        out_cp(i - 2, s).wait()
                    transpose_rows(slabs[s], obufs[s], PR)
                    out_cp(i, s).start()

                    @pl.when(i + 2 < nch)
                    def _():
                        in_cp(i + 2, s).start()
            return carry

        lax.fori_loop(0, (base_n + 2) // 2, body2, 0, unroll=False)
        for s in range(2):
            last = jnp.where((nch - 1) % 2 == s, nch - 1, nch - 2)
            out_cp(last, s).wait()

        if V % W:
            @pl.when(wid == 0)
            def _():
                pltpu.sync_copy(tail_hbm, tl_v.at[:, pl.ds(0, 128)])
                transpose_rows(tl_v, ob0, 64)
                pltpu.sync_copy(ob0.at[pl.ds(0, 64)],
                                out_hbm.at[pl.ds(V // 2 - 64, 64)])

    return pack_kernel(tableT, tailT)


def _make_gather(B, T, V, D, NC, NS):
    NW = NC * NS
    bpw = B // NW
    npr = T // 2
    groups = npr // 2
    gr = 4 * bpw
    nvec = gr // 16
    assert B % NW == 0 and T % 4 == 0 and groups % 2 == 0 and bpw == 128

    mesh = plsc.VectorSubcoreMesh(core_axis_name="c", subcore_axis_name="s")

    @functools.partial(
        pl.kernel,
        out_type=jax.ShapeDtypeStruct((B * T, D), jnp.float32),
        mesh=mesh,
        compiler_params=pltpu.CompilerParams(
            use_tc_tiling_on_sc=False, skip_device_barrier=True,
            needs_layout_passes=False,
        ),
        scratch_types=[
            pltpu.VMEM((bpw * T,), jnp.int32),
            pltpu.VMEM((gr,), jnp.int32),
            pltpu.VMEM((gr,), jnp.int32),
            pltpu.VMEM((gr,), jnp.int32),
            pltpu.VMEM((gr, D), jnp.float32),
            pltpu.VMEM((gr, D), jnp.float32),
            pltpu.SemaphoreType.DMA,
            pltpu.SemaphoreType.DMA,
        ],
    )
    def gather_kernel(ids_hbm, table_hbm, out_hbm, idx_v, off_v,
                      ig0, ig1, b0, b1, g0, g1):
        igs = (ig0, ig1)
        bufs = (b0, b1)
        gsems = (g0, g1)
        wid = lax.axis_index("s") * NC + lax.axis_index("c")
        wrow = wid * bpw * T
        pltpu.sync_copy(ids_hbm.at[pl.ds(pl.multiple_of(wrow, 8), bpw * T)],
                        idx_v)

        for m in range(nvec):
            k = lax.broadcasted_iota(jnp.int32, (16,), 0) + (m * 16)
            jl = lax.shift_right_logical(k, 8)
            b = lax.shift_right_logical(k & (2 * bpw - 1), 1)
            h = k & 1
            off_v[pl.ds(m * 16, 16)] = b * T + 2 * jl + h

        full = (V // 256) * 256

        def build_idx(g, s):
            tadd = 4 * g
            for m in range(nvec):
                ov = off_v[pl.ds(m * 16, 16)] + tadd
                v = plsc.load_gather(idx_v, [ov])
                pi = (v & (-256)) + ((v & 127) << 1) + (
                    lax.shift_right_logical(v, 7) & 1)
                igs[s][pl.ds(m * 16, 16)] = jnp.where(v < full, pi, v)

        def gather_cp(s):
            return pltpu.make_async_copy(
                table_hbm.at[igs[s]], bufs[s], gsems[s])

        def out_write(g, s):
            start = pl.multiple_of(wrow + g * gr, 8)
            pltpu.sync_copy(bufs[s], out_hbm.at[pl.ds(start, gr)])

        build_idx(0, 0)
        gather_cp(0).start()
        build_idx(1, 1)
        gather_cp(1).start()

        def body(m, carry):
            g = 2 * m
            gather_cp(0).wait()
            out_write(g, 0)
            build_idx(g + 2, 0)
            gather_cp(0).start()
            gather_cp(1).wait()
            out_write(g + 1, 1)
            build_idx(g + 3, 1)
            gather_cp(1).start()
            return carry

        lax.fori_loop(0, groups // 2 - 1, body, 0, unroll=False)

        gather_cp(0).wait()
        out_write(groups - 2, 0)
        gather_cp(1).wait()
        out_write(groups - 1, 1)

    return gather_kernel


def _unpack_body(z_ref, o_ref):
    npr = o_ref.shape[0] // 2
    for j in range(npr):
        slab = z_ref[j * 128:(j + 1) * 128, :]
        st = _mxu_t(slab)
        o_ref[2 * j:2 * j + 2, :, :] = st.reshape(2, o_ref.shape[1], 128)


def _unpack(z128, B, T, D):
    nb = B // 128
    rows = T // 2 * 128
    return pl.pallas_call(
        _unpack_body,
        grid=(nb,),
        in_specs=[pl.BlockSpec((rows, 128), lambda i: (i, 0))],
        out_specs=pl.BlockSpec((T, D, 128), lambda i: (0, 0, i)),
        out_shape=jax.ShapeDtypeStruct((T, D, B), jnp.float32),
    )(z128)


def _mask_body(ids_ref, m_ref):
    m_ref[...] = ids_ref[...] != 0


def kernel(ids, table):
    B, T = ids.shape
    V, D = table.shape

    info = plsc.get_sparse_core_info()
    NC, NS = info.num_cores, info.num_subcores

    packed = _tc_pack_table(table.T)
    table_lin = packed.reshape(V, D)

    gather = _make_gather(B, T, V, D, NC, NS)
    z2 = gather(ids.reshape(B * T), table_lin)
    out3 = _unpack(z2.reshape(B * T // 2, 128), B, T, D)
    tokens = jnp.transpose(out3, (2, 0, 1))

    mask = pl.pallas_call(
        _mask_body,
        out_shape=jax.ShapeDtypeStruct((B, T), jnp.bool_),
    )(ids)
    return tokens, mask


# speedup vs baseline: 4.4875x; 1.1595x over previous
import functools

import jax
import jax.numpy as jnp
from jax import lax
from jax.experimental import pallas as pl
from jax.experimental.pallas import tpu as pltpu
from jax.experimental.pallas import tpu_sc as plsc


def _eye(n):
    a = lax.broadcasted_iota(jnp.int32, (n, n), 0)
    b = lax.broadcasted_iota(jnp.int32, (n, n), 1)
    return (a == b).astype(jnp.float32)


def _mxu_t(x):
    return lax.dot_general(
        x, _eye(x.shape[0]), (((0,), (0,)), ((), ())),
        preferred_element_type=jnp.float32)


def _tc_pack_body(tT_ref, o_ref):
    D = tT_ref.shape[0]
    W = tT_ref.shape[1]
    i = pl.program_id(0)
    nb = pl.num_programs(0)

    @pl.when(i < nb - 1)
    def _():
        x = tT_ref[...]
        for k in range(W // 256):
            m = jnp.concatenate(
                [x[:, k * 256:k * 256 + 128],
                 x[:, k * 256 + 128:k * 256 + 256]], axis=0)
            o_ref[k * 128:(k + 1) * 128, :] = _mxu_t(m)

    @pl.when(i == nb - 1)
    def _():
        x = tT_ref[:, 0:64]
        xt = _mxu_t(x)
        y = xt.reshape(32, 2, D)
        o_ref[0:32, :] = jnp.concatenate([y[:, 0, :], y[:, 1, :]], axis=1)


def _tc_pack_table(tableT, W=10752):
    D, V = tableT.shape
    full = (V // 256) * 256
    nb = full // W + 1
    assert full % W == 0 and W % 256 == 0
    return pl.pallas_call(
        _tc_pack_body,
        grid=(nb,),
        in_specs=[pl.BlockSpec((D, W), lambda i: (0, i))],
        out_specs=pl.BlockSpec((W * D // 128, 128), lambda i: (i, 0)),
        out_shape=jax.ShapeDtypeStruct((V * D // 128, 128), jnp.float32),
    )(tableT)


def _make_gather(B, T, V, D, NC, NS):
    NW = NC * NS
    bpw = B // NW
    npr = T // 2
    groups = npr // 2
    gr = 4 * bpw
    nvec = gr // 16
    assert B % NW == 0 and T % 4 == 0 and groups % 2 == 0 and bpw == 128

    mesh = plsc.VectorSubcoreMesh(core_axis_name="c", subcore_axis_name="s")

    @functools.partial(
        pl.kernel,
        out_type=jax.ShapeDtypeStruct((B * T, D), jnp.float32),
        mesh=mesh,
        compiler_params=pltpu.CompilerParams(
            use_tc_tiling_on_sc=False, skip_device_barrier=True,
            needs_layout_passes=False,
        ),
        scratch_types=[
            pltpu.VMEM((bpw * T,), jnp.int32),
            pltpu.VMEM((gr,), jnp.int32),
            pltpu.VMEM((gr,), jnp.int32),
            pltpu.VMEM((gr,), jnp.int32),
            pltpu.VMEM((gr, D), jnp.float32),
            pltpu.VMEM((gr, D), jnp.float32),
            pltpu.SemaphoreType.DMA,
            pltpu.SemaphoreType.DMA,
        ],
    )
    def gather_kernel(ids_hbm, table_hbm, out_hbm, idx_v, off_v,
                      ig0, ig1, b0, b1, g0, g1):
        igs = (ig0, ig1)
        bufs = (b0, b1)
        gsems = (g0, g1)
        wid = lax.axis_index("s") * NC + lax.axis_index("c")
        wrow = wid * bpw * T
        pltpu.sync_copy(ids_hbm.at[pl.ds(pl.multiple_of(wrow, 8), bpw * T)],
                        idx_v)

        for m in range(nvec):
            k = lax.broadcasted_iota(jnp.int32, (16,), 0) + (m * 16)
            jl = lax.shift_right_logical(k, 8)
            b = lax.shift_right_logical(k & (2 * bpw - 1), 1)
            h = k & 1
            off_v[pl.ds(m * 16, 16)] = b * T + 2 * jl + h

        full = (V // 256) * 256

        def build_idx(g, s):
            tadd = 4 * g
            for m in range(nvec):
                ov = off_v[pl.ds(m * 16, 16)] + tadd
                v = plsc.load_gather(idx_v, [ov])
                pi = (v & (-256)) + ((v & 127) << 1) + (
                    lax.shift_right_logical(v, 7) & 1)
                igs[s][pl.ds(m * 16, 16)] = jnp.where(v < full, pi, v)

        def gather_cp(s):
            return pltpu.make_async_copy(
                table_hbm.at[igs[s]], bufs[s], gsems[s])

        def out_write(g, s):
            start = pl.multiple_of(wrow + g * gr, 8)
            pltpu.sync_copy(bufs[s], out_hbm.at[pl.ds(start, gr)])

        build_idx(0, 0)
        gather_cp(0).start()
        build_idx(1, 1)
        gather_cp(1).start()

        def body(m, carry):
            g = 2 * m
            gather_cp(0).wait()
            out_write(g, 0)
            build_idx(g + 2, 0)
            gather_cp(0).start()
            gather_cp(1).wait()
            out_write(g + 1, 1)
            build_idx(g + 3, 1)
            gather_cp(1).start()
            return carry

        lax.fori_loop(0, groups // 2 - 1, body, 0, unroll=False)

        gather_cp(0).wait()
        out_write(groups - 2, 0)
        gather_cp(1).wait()
        out_write(groups - 1, 1)

    return gather_kernel


def _unpack_body(z_ref, o_ref):
    npr = o_ref.shape[0] // 2
    for j in range(npr):
        slab = z_ref[j * 128:(j + 1) * 128, :]
        st = _mxu_t(slab)
        o_ref[2 * j:2 * j + 2, :, :] = st.reshape(2, o_ref.shape[1], 128)


def _unpack(z128, B, T, D):
    nb = B // 128
    rows = T // 2 * 128
    return pl.pallas_call(
        _unpack_body,
        grid=(nb,),
        in_specs=[pl.BlockSpec((rows, 128), lambda i: (i, 0))],
        out_specs=pl.BlockSpec((T, D, 128), lambda i: (0, 0, i)),
        out_shape=jax.ShapeDtypeStruct((T, D, B), jnp.float32),
    )(z128)


def _mask_body(ids_ref, m_ref):
    m_ref[...] = ids_ref[...] != 0


def kernel(ids, table):
    B, T = ids.shape
    V, D = table.shape

    info = plsc.get_sparse_core_info()
    NC, NS = info.num_cores, info.num_subcores

    packed = _tc_pack_table(table.T)
    table_lin = packed.reshape(V, D)

    gather = _make_gather(B, T, V, D, NC, NS)
    z2 = gather(ids.reshape(B * T), table_lin)
    out3 = _unpack(z2.reshape(B * T // 2, 128), B, T, D)
    tokens = jnp.transpose(out3, (2, 0, 1))

    mask = pl.pallas_call(
        _mask_body,
        out_shape=jax.ShapeDtypeStruct((B, T), jnp.bool_),
    )(ids)
    return tokens, mask
